# Initial kernel scaffold; baseline (speedup 1.0000x reference)
import jax
import jax.numpy as jnp
from jax.experimental import pallas as pl


def kernel(state, node_feature, edge_feature, edge_index, params):
    raise NotImplementedError("write your pallas kernel here")


# speedup vs baseline: 1.0616x; 1.0616x over previous
import functools

import jax
import jax.numpy as jnp
from jax.experimental import pallas as pl
from jax.experimental.pallas import tpu as pltpu

N = 10000
E = 160000
H = 256
DS = 64
ITERS = 2

RB = 512
EB = 1024
NPAD = 10240
EPAD = 157 * 1024


def _ln(x, g, t):
    mu = jnp.mean(x, axis=-1, keepdims=True)
    var = jnp.mean((x - mu) ** 2, axis=-1, keepdims=True)
    return (x - mu) * jax.lax.rsqrt(var + 1e-5) * g + t


def _trunk(x, W1, b1, g1, t1, W2, b2, g2, t2):
    h = x @ W1 + b1
    h = jnp.maximum(_ln(h, g1, t1), 0.0)
    h = h @ W2 + b2
    h = jnp.maximum(_ln(h, g2, t2), 0.0)
    return h


def _mlp_body(x_ref, W1, b1, g1, t1, W2, b2, g2, t2, Wo, bo, o_ref, *, act):
    h = _trunk(x_ref[...], W1[...], b1[...], g1[...], t1[...],
               W2[...], b2[...], g2[...], t2[...])
    y = h @ Wo[...] + bo[...]
    if act == "tanh":
        y = jnp.tanh(y)
    o_ref[...] = y


def _mlp_call(x, p, W1, dout, rb, act=None):
    rows = x.shape[0]
    k = x.shape[1]
    full = lambda a: pl.BlockSpec(a.shape, lambda i: (0,) * a.ndim)
    args = (W1, p['b1'], p['g1'], p['t1'], p['W2'], p['b2'], p['g2'], p['t2'],
            p['Wo'], p['bo'])
    return pl.pallas_call(
        functools.partial(_mlp_body, act=act),
        grid=(rows // rb,),
        in_specs=[pl.BlockSpec((rb, k), lambda i: (i, 0))] + [full(a) for a in args],
        out_specs=pl.BlockSpec((rb, dout), lambda i: (i, 0)),
        out_shape=jax.ShapeDtypeStruct((rows, dout), jnp.float32),
        compiler_params=pltpu.CompilerParams(
            dimension_semantics=("parallel",)),
    )(x, *args)


def _head_body(h_ref, W1, b1, g1, t1, W2, b2, g2, t2, Wo2, bo2, o_ref):
    hid = _trunk(h_ref[...], W1[...], b1[...], g1[...], t1[...],
                 W2[...], b2[...], g2[...], t2[...])
    y = hid @ Wo2[...] + bo2[...]
    o = jnp.concatenate([jnp.tanh(y[:, 0:1]), jax.nn.sigmoid(y[:, 1:2])], axis=1)
    nrm = jnp.sqrt(jnp.sum(o * o, axis=1, keepdims=True))
    o_ref[...] = o / jnp.maximum(nrm, 1e-12)


def _head_call(h, p):
    Wo2 = jnp.concatenate([p['Wo'], p['Ws']], axis=1)
    bo2 = jnp.concatenate([p['bo'], p['bs']], axis=0)
    full = lambda a: pl.BlockSpec(a.shape, lambda i: (0,) * a.ndim)
    args = (p['W1'], p['b1'], p['g1'], p['t1'], p['W2'], p['b2'], p['g2'],
            p['t2'], Wo2, bo2)
    return pl.pallas_call(
        _head_body,
        grid=(NPAD // RB,),
        in_specs=[pl.BlockSpec((RB, DS), lambda i: (i, 0))] + [full(a) for a in args],
        out_specs=pl.BlockSpec((RB, 2), lambda i: (i, 0)),
        out_shape=jax.ShapeDtypeStruct((NPAD, 2), jnp.float32),
        compiler_params=pltpu.CompilerParams(
            dimension_semantics=("parallel",)),
    )(h, *args)


def kernel(state, node_feature, edge_feature, edge_index, params):
    src = edge_index[0]
    dst = edge_index[1]

    glob = jnp.broadcast_to(state[0, :5], (N, 5))
    loc1 = state[0, 5:5 + N][:, None]
    loc2 = state[0, 5 + N:5 + 2 * N][:, None]
    x13 = jnp.concatenate([glob, loc1, loc2, node_feature], axis=1)
    xpad = jnp.pad(x13, ((0, NPAD - N), (0, 3)))

    pin = params['input']
    W1in = jnp.pad(pin['W1'], ((0, 3), (0, 0)))
    h = _mlp_call(xpad, pin, W1in, DS, RB)

    pm = params['message']
    W1g = jnp.concatenate([
        pm['W1'][0:64], pm['W1'][65:78], pm['W1'][64:65],
        jnp.zeros((2, H), jnp.float32)], axis=0)

    pu = params['update']
    ef = edge_feature[:, None]
    dstp = jnp.where(jnp.arange(EPAD) < E,
                     jnp.pad(dst, (0, EPAD - E)), N)

    for _ in range(ITERS):
        hx = jnp.concatenate([h[:N], x13], axis=1)
        g = jnp.take(hx, src, axis=0)
        g = jnp.concatenate([g, ef], axis=1)
        g = jnp.pad(g, ((0, EPAD - E), (0, 2)))
        m = _mlp_call(g, pm, W1g, DS, EB, act="tanh")
        m_hat = jax.ops.segment_max(m, dstp, num_segments=N)
        m_hat = jnp.where(jnp.isneginf(m_hat), 0.0, m_hat)
        ui = jnp.pad(jnp.concatenate([m_hat, h[:N]], axis=1),
                     ((0, NPAD - N), (0, 0)))
        h = _mlp_call(ui, pu, pu['W1'], DS, RB)

    out = _head_call(h, params['output'])
    return out[:N].reshape(N, 1, 2)


# speedup vs baseline: 1.4512x; 1.3670x over previous
import functools

import jax
import jax.numpy as jnp
from jax import lax
from jax.experimental import pallas as pl
from jax.experimental.pallas import tpu as pltpu
from jax.experimental.pallas import tpu_sc as plsc

N = 10000
E = 160000
H = 256
DS = 64
ITERS = 2

RB = 512
EB = 1024
NPAD = 10240
EPAD = 163840

NW = 32
SH = 313
NSEG = NW * SH
TRASH = SH
CAP = EPAD
CHUNK = 4096
GB = 512
NEG = -3.0e38
TRASH_PACKED = TRASH << 18

_mesh = plsc.VectorSubcoreMesh(core_axis_name="c", subcore_axis_name="s")


def _wid():
    return lax.axis_index("s") * 2 + lax.axis_index("c")


def _iota():
    return jnp.arange(16, dtype=jnp.int32)


def _ln(x, g, t):
    mu = jnp.mean(x, axis=-1, keepdims=True)
    var = jnp.mean((x - mu) ** 2, axis=-1, keepdims=True)
    return (x - mu) * lax.rsqrt(var + 1e-5) * g + t


def _trunk(x, W1, b1, g1, t1, W2, b2, g2, t2):
    h = x @ W1 + b1
    h = jnp.maximum(_ln(h, g1, t1), 0.0)
    h = h @ W2 + b2
    h = jnp.maximum(_ln(h, g2, t2), 0.0)
    return h


def _mlp_body(x_ref, W1, b1, g1, t1, W2, b2, g2, t2, Wo, bo, o_ref):
    h = _trunk(x_ref[...], W1[...], b1[...], g1[...], t1[...],
               W2[...], b2[...], g2[...], t2[...])
    o_ref[...] = h @ Wo[...] + bo[...]


def _mlp_call(x, p, W1, dout, rb):
    rows, k = x.shape
    full = lambda a: pl.BlockSpec(a.shape, lambda i: (0,) * a.ndim)
    args = (W1, p['b1'], p['g1'], p['t1'], p['W2'], p['b2'], p['g2'], p['t2'],
            p['Wo'], p['bo'])
    return pl.pallas_call(
        _mlp_body,
        grid=(rows // rb,),
        in_specs=[pl.BlockSpec((rb, k), lambda i: (i, 0))] + [full(a) for a in args],
        out_specs=pl.BlockSpec((rb, dout), lambda i: (i, 0)),
        out_shape=jax.ShapeDtypeStruct((rows, dout), jnp.float32),
        compiler_params=pltpu.CompilerParams(dimension_semantics=("parallel",)),
    )(x, *args)


def _edge_body(g_ref, ef_ref, W1, b1, g1, t1, W2, b2, g2, t2, Wo, bo, we,
               o_ref):
    x1 = g_ref[...] @ W1[...] + b1[...] + ef_ref[...] * we[...]
    h = jnp.maximum(_ln(x1, g1[...], t1[...]), 0.0)
    h = h @ W2[...] + b2[...]
    h = jnp.maximum(_ln(h, g2[...], t2[...]), 0.0)
    o_ref[...] = jnp.tanh(h @ Wo[...] + bo[...])


def _edge_mlp(g, ef, p, W1g, we):
    full = lambda a: pl.BlockSpec(a.shape, lambda i: (0,) * a.ndim)
    Wop = jnp.pad(p['Wo'], ((0, 0), (0, 128 - DS)))
    bop = jnp.pad(p['bo'], (0, 128 - DS))
    args = (W1g, p['b1'], p['g1'], p['t1'], p['W2'], p['b2'], p['g2'],
            p['t2'], Wop, bop, we)
    return pl.pallas_call(
        _edge_body,
        grid=(EPAD // EB,),
        in_specs=[pl.BlockSpec((EB, 128), lambda i: (i, 0)),
                  pl.BlockSpec((EB, 1), lambda i: (i, 0))]
                 + [full(a) for a in args],
        out_specs=pl.BlockSpec((EB, 128), lambda i: (i, 0)),
        out_shape=jax.ShapeDtypeStruct((EPAD, 128), jnp.float32),
        compiler_params=pltpu.CompilerParams(dimension_semantics=("parallel",)),
    )(g, ef, *args)


def _head_body(h_ref, W1, b1, g1, t1, W2, b2, g2, t2, Wo2, bo2, o_ref):
    hid = _trunk(h_ref[...], W1[...], b1[...], g1[...], t1[...],
                 W2[...], b2[...], g2[...], t2[...])
    y = hid @ Wo2[...] + bo2[...]
    o = jnp.concatenate([jnp.tanh(y[:, 0:1]), jax.nn.sigmoid(y[:, 1:2])],
                        axis=1)
    nrm = jnp.sqrt(jnp.sum(o * o, axis=1, keepdims=True))
    o_ref[...] = o / jnp.maximum(nrm, 1e-12)


def _head_call(h, p):
    Wo2 = jnp.concatenate([p['Wo'], p['Ws']], axis=1)
    bo2 = jnp.concatenate([p['bo'], p['bs']], axis=0)
    full = lambda a: pl.BlockSpec(a.shape, lambda i: (0,) * a.ndim)
    args = (p['W1'], p['b1'], p['g1'], p['t1'], p['W2'], p['b2'], p['g2'],
            p['t2'], Wo2, bo2)
    return pl.pallas_call(
        _head_body,
        grid=(NPAD // RB,),
        in_specs=[pl.BlockSpec((RB, DS), lambda i: (i, 0))] + [full(a) for a in args],
        out_specs=pl.BlockSpec((RB, 2), lambda i: (i, 0)),
        out_shape=jax.ShapeDtypeStruct((NPAD, 2), jnp.float32),
        compiler_params=pltpu.CompilerParams(dimension_semantics=("parallel",)),
    )(h, *args)


def _prep_body(dst_hbm, lists_hbm, counts_hbm, dbuf, lbuf, cbuf):
    w = _wid()
    lo = w * SH
    hi = lo + SH
    iota = _iota()

    def chunk_body(ch, carry):
        mcnt, nflush = carry
        off = pl.multiple_of(ch * CHUNK, CHUNK)
        pltpu.sync_copy(dst_hbm.at[pl.ds(off, CHUNK)], dbuf)

        def group_body(g, mcnt):
            dvec = dbuf[pl.ds(g * 16, 16)]
            msk = (dvec >= lo) & (dvec < hi)
            mi = msk.astype(jnp.int32)
            excl = plsc.cumsum(mi) - mi
            packed = ((dvec - lo) << 18) | (off + g * 16 + iota)
            plsc.store_scatter(lbuf, [excl + mcnt], packed, mask=msk)
            return mcnt + jnp.sum(mi)

        mcnt = lax.fori_loop(0, CHUNK // 16, group_body, mcnt)

        def do_flush(c):
            mc, nf = c
            foff = pl.multiple_of(nf * CHUNK, CHUNK)
            pltpu.sync_copy(lbuf.at[pl.ds(0, CHUNK)],
                            lists_hbm.at[w, pl.ds(foff, CHUNK)])

            def mv(i, _):
                v = lbuf[pl.ds(CHUNK + i * 16, 16)]
                lbuf[pl.ds(i * 16, 16)] = v
                return 0

            lax.fori_loop(0, CHUNK // 16, mv, 0)
            return (mc - CHUNK, nf + 1)

        return lax.cond(mcnt >= CHUNK, do_flush, lambda c: c, (mcnt, nflush))

    mcnt, nflush = lax.fori_loop(0, EPAD // CHUNK, chunk_body, (0, 0))
    foff = pl.multiple_of(nflush * CHUNK, CHUNK)
    pltpu.sync_copy(lbuf.at[pl.ds(0, CHUNK)], lists_hbm.at[w, pl.ds(foff, CHUNK)])
    cbuf[...] = jnp.full((16,), nflush * CHUNK + mcnt, jnp.int32)
    pltpu.sync_copy(cbuf, counts_hbm.at[w])


def _prep_call(dstp):
    return pl.kernel(
        _prep_body,
        compiler_params=pltpu.CompilerParams(needs_layout_passes=False),
        out_type=(jax.ShapeDtypeStruct((NW, CAP), jnp.int32),
                  jax.ShapeDtypeStruct((NW, 16), jnp.int32)),
        mesh=_mesh,
        scratch_types=[pltpu.VMEM((CHUNK,), jnp.int32),
                       pltpu.VMEM((2 * CHUNK + 32,), jnp.int32),
                       pltpu.VMEM((16,), jnp.int32)],
    )(dstp)


def _gather_body(table_hbm, src_hbm, g_hbm, ibuf, rows, sem):
    w = _wid()
    base = w * (EPAD // NW)

    def batch(b, _):
        r0 = pl.multiple_of(base + b * GB, GB)
        for j in range(4):
            pltpu.sync_copy(src_hbm.at[pl.ds(r0 + j * 128, 128)], ibuf.at[j])
        cps = [pltpu.async_copy(table_hbm.at[ibuf.at[j]],
                                rows.at[pl.ds(j * 128, 128)], sem)
               for j in range(4)]
        for c in cps:
            c.wait()
        pltpu.sync_copy(rows, g_hbm.at[pl.ds(r0, GB)])
        return 0

    lax.fori_loop(0, EPAD // NW // GB, batch, 0)


def _gather_call(table, srcp):
    return pl.kernel(
        _gather_body,
        compiler_params=pltpu.CompilerParams(needs_layout_passes=False),
        out_type=jax.ShapeDtypeStruct((EPAD, 128), jnp.float32),
        mesh=_mesh,
        scratch_types=[pltpu.VMEM((4, 128), jnp.int32),
                       pltpu.VMEM((GB, 128), jnp.float32),
                       pltpu.SemaphoreType.DMA],
    )(table, srcp)


def _scat_body(m_hbm, lists_hbm, counts_hbm, mhat_hbm,
               acc, lbatch, dv, eidx, rows, cbuf, sem):
    w = _wid()
    iota = _iota()
    offk = [iota + 16 * k for k in range(4)]

    def init(i, _):
        acc[pl.ds(i * 16, 16)] = jnp.full((16,), NEG, jnp.float32)
        return 0

    lax.fori_loop(0, (SH + 1) * DS // 16, init, 0)

    pltpu.sync_copy(counts_hbm.at[w], cbuf)
    nt = jnp.max(cbuf[...])
    nbatch = (nt + GB - 1) // GB

    def batch(b, _):
        boff = pl.multiple_of(b * GB, GB)
        pltpu.sync_copy(lists_hbm.at[w, pl.ds(boff, GB)], lbatch)

        def unpack(g, _):
            p = lbatch[pl.ds(g * 16, 16)]
            glob = b * GB + g * 16 + iota
            p = jnp.where(glob < nt, p, TRASH_PACKED)
            e = p & 0x3FFFF
            dl = p >> 18
            j = g // 8
            c0 = (g % 8) * 16
            plsc.store_scatter(eidx, [jnp.full((16,), j, jnp.int32),
                                      c0 + iota], e)
            dv[pl.ds(g * 16, 16)] = dl
            return 0

        lax.fori_loop(0, GB // 16, unpack, 0)

        cps = [pltpu.async_copy(m_hbm.at[eidx.at[j]],
                                rows.at[pl.ds(j * 128, 128)], sem)
               for j in range(4)]
        for c in cps:
            c.wait()

        def rmw(g, _):
            dvec = dv[pl.ds(g * 16, 16)]
            for e in range(16):
                el = jnp.full((16,), g * 16 + e, jnp.int32)
                dl = plsc.load_gather(dv, [el])
                basev = dl * DS
                for k in range(4):
                    idx = basev + offk[k]
                    row = plsc.load_gather(rows, [el, offk[k]])
                    cur = plsc.load_gather(acc, [idx])
                    plsc.store_scatter(acc, [idx], jnp.maximum(cur, row))
            return 0

        lax.fori_loop(0, GB // 16, rmw, 0)
        return 0

    lax.fori_loop(0, nbatch, batch, 0)

    def fixup(i, _):
        v = acc[pl.ds(i * 16, 16)]
        acc[pl.ds(i * 16, 16)] = jnp.where(v == NEG, 0.0, v)
        return 0

    lax.fori_loop(0, SH * DS // 16, fixup, 0)
    pltpu.sync_copy(acc.at[pl.ds(0, SH * DS)],
                    mhat_hbm.at[pl.ds(w * SH * DS, SH * DS)])


def _scatter_call(m, lists, counts):
    return pl.kernel(
        _scat_body,
        compiler_params=pltpu.CompilerParams(needs_layout_passes=False),
        out_type=jax.ShapeDtypeStruct((NSEG * DS,), jnp.float32),
        mesh=_mesh,
        scratch_types=[pltpu.VMEM(((SH + 1) * DS,), jnp.float32),
                       pltpu.VMEM((GB,), jnp.int32),
                       pltpu.VMEM((GB,), jnp.int32),
                       pltpu.VMEM((4, 128), jnp.int32),
                       pltpu.VMEM((GB, 128), jnp.float32),
                       pltpu.VMEM((16,), jnp.int32),
                       pltpu.SemaphoreType.DMA],
    )(m, lists, counts)


def kernel(state, node_feature, edge_feature, edge_index, params):
    src = edge_index[0]
    dst = edge_index[1]

    glob = jnp.broadcast_to(state[0, :5], (N, 5))
    loc1 = state[0, 5:5 + N][:, None]
    loc2 = state[0, 5 + N:5 + 2 * N][:, None]
    x13 = jnp.concatenate([glob, loc1, loc2, node_feature], axis=1)
    xpad = jnp.pad(x13, ((0, NPAD - N), (0, 3)))

    pin = params['input']
    W1in = jnp.pad(pin['W1'], ((0, 3), (0, 0)))
    h = _mlp_call(xpad, pin, W1in, DS, RB)

    pm = params['message']
    W1g = jnp.concatenate([pm['W1'][0:64], pm['W1'][65:78],
                           jnp.zeros((51, H), jnp.float32)], axis=0)
    we = pm['W1'][64:65]
    pu = params['update']

    srcp = jnp.pad(src, (0, EPAD - E))
    dstp = jnp.pad(dst, (0, EPAD - E), constant_values=1 << 20)
    efp = jnp.pad(edge_feature, (0, EPAD - E))[:, None]

    lists, counts = _prep_call(dstp)

    for _ in range(ITERS):
        table = jnp.concatenate(
            [h[:N], x13, jnp.zeros((N, 51), jnp.float32)], axis=1)
        g = _gather_call(table, srcp)
        m = _edge_mlp(g, efp, pm, W1g, we)
        mh = _scatter_call(m, lists, counts)
        m_hat = mh.reshape(NSEG, DS)[:N]
        ui = jnp.pad(jnp.concatenate([m_hat, h[:N]], axis=1),
                     ((0, NPAD - N), (0, 0)))
        h = _mlp_call(ui, pu, pu['W1'], DS, RB)

    out = _head_call(h, params['output'])
    return out[:N].reshape(N, 1, 2)


# speedup vs baseline: 1.5356x; 1.0582x over previous
import functools

import jax
import jax.numpy as jnp
from jax import lax
from jax.experimental import pallas as pl
from jax.experimental.pallas import tpu as pltpu
from jax.experimental.pallas import tpu_sc as plsc

N = 10000
E = 160000
H = 256
DS = 64
ITERS = 2

RB = 512
EB = 1024
NPAD = 10240
EPAD = 163840

NW = 32
SH = 313
NSEG = NW * SH
TRASH = SH
CAP = EPAD
CHUNK = 4096
GB = 256
NEG = -3.0e38
TRASH_PACKED = TRASH << 18

_mesh = plsc.VectorSubcoreMesh(core_axis_name="c", subcore_axis_name="s")


def _wid():
    return lax.axis_index("s") * 2 + lax.axis_index("c")


def _iota():
    return jnp.arange(16, dtype=jnp.int32)


def _ln(x, g, t):
    mu = jnp.mean(x, axis=-1, keepdims=True)
    var = jnp.mean((x - mu) ** 2, axis=-1, keepdims=True)
    return (x - mu) * lax.rsqrt(var + 1e-5) * g + t


def _trunk(x, W1, b1, g1, t1, W2, b2, g2, t2):
    h = x @ W1 + b1
    h = jnp.maximum(_ln(h, g1, t1), 0.0)
    h = h @ W2 + b2
    h = jnp.maximum(_ln(h, g2, t2), 0.0)
    return h


def _mlp_body(x_ref, W1, b1, g1, t1, W2, b2, g2, t2, Wo, bo, o_ref):
    h = _trunk(x_ref[...], W1[...], b1[...], g1[...], t1[...],
               W2[...], b2[...], g2[...], t2[...])
    o_ref[...] = h @ Wo[...] + bo[...]


def _mlp_call(x, p, W1, dout, rb):
    rows, k = x.shape
    full = lambda a: pl.BlockSpec(a.shape, lambda i: (0,) * a.ndim)
    args = (W1, p['b1'], p['g1'], p['t1'], p['W2'], p['b2'], p['g2'], p['t2'],
            p['Wo'], p['bo'])
    return pl.pallas_call(
        _mlp_body,
        grid=(rows // rb,),
        in_specs=[pl.BlockSpec((rb, k), lambda i: (i, 0))] + [full(a) for a in args],
        out_specs=pl.BlockSpec((rb, dout), lambda i: (i, 0)),
        out_shape=jax.ShapeDtypeStruct((rows, dout), jnp.float32),
        compiler_params=pltpu.CompilerParams(dimension_semantics=("parallel",)),
    )(x, *args)


def _edge_body(g_ref, ef_ref, W1, b1, g1, t1, W2, b2, g2, t2, Wo, bo, we,
               o_ref):
    x1 = g_ref[...] @ W1[...] + b1[...] + ef_ref[...] * we[...]
    h = jnp.maximum(_ln(x1, g1[...], t1[...]), 0.0)
    h = h @ W2[...] + b2[...]
    h = jnp.maximum(_ln(h, g2[...], t2[...]), 0.0)
    o_ref[...] = jnp.tanh(h @ Wo[...] + bo[...])


def _edge_mlp(g, ef, p, W1g, we):
    full = lambda a: pl.BlockSpec(a.shape, lambda i: (0,) * a.ndim)
    Wop = jnp.pad(p['Wo'], ((0, 0), (0, 128 - DS)))
    bop = jnp.pad(p['bo'], (0, 128 - DS))
    args = (W1g, p['b1'], p['g1'], p['t1'], p['W2'], p['b2'], p['g2'],
            p['t2'], Wop, bop, we)
    return pl.pallas_call(
        _edge_body,
        grid=(EPAD // EB,),
        in_specs=[pl.BlockSpec((EB, 128), lambda i: (i, 0)),
                  pl.BlockSpec((EB, 1), lambda i: (i, 0))]
                 + [full(a) for a in args],
        out_specs=pl.BlockSpec((EB, 128), lambda i: (i, 0)),
        out_shape=jax.ShapeDtypeStruct((EPAD, 128), jnp.float32),
        compiler_params=pltpu.CompilerParams(dimension_semantics=("parallel",)),
    )(g, ef, *args)


def _head_body(h_ref, W1, b1, g1, t1, W2, b2, g2, t2, Wo2, bo2, o_ref):
    hid = _trunk(h_ref[...], W1[...], b1[...], g1[...], t1[...],
                 W2[...], b2[...], g2[...], t2[...])
    y = hid @ Wo2[...] + bo2[...]
    o = jnp.concatenate([jnp.tanh(y[:, 0:1]), jax.nn.sigmoid(y[:, 1:2])],
                        axis=1)
    nrm = jnp.sqrt(jnp.sum(o * o, axis=1, keepdims=True))
    o_ref[...] = o / jnp.maximum(nrm, 1e-12)


def _head_call(h, p):
    Wo2 = jnp.concatenate([p['Wo'], p['Ws']], axis=1)
    bo2 = jnp.concatenate([p['bo'], p['bs']], axis=0)
    full = lambda a: pl.BlockSpec(a.shape, lambda i: (0,) * a.ndim)
    args = (p['W1'], p['b1'], p['g1'], p['t1'], p['W2'], p['b2'], p['g2'],
            p['t2'], Wo2, bo2)
    return pl.pallas_call(
        _head_body,
        grid=(NPAD // RB,),
        in_specs=[pl.BlockSpec((RB, DS), lambda i: (i, 0))] + [full(a) for a in args],
        out_specs=pl.BlockSpec((RB, 2), lambda i: (i, 0)),
        out_shape=jax.ShapeDtypeStruct((NPAD, 2), jnp.float32),
        compiler_params=pltpu.CompilerParams(dimension_semantics=("parallel",)),
    )(h, *args)


def _prep_body(dst_hbm, lists_hbm, counts_hbm, dbuf, lbuf, cbuf):
    w = _wid()
    lo = w * SH
    hi = lo + SH
    iota = _iota()

    def chunk_body(ch, carry):
        mcnt, nflush = carry
        off = pl.multiple_of(ch * CHUNK, CHUNK)
        pltpu.sync_copy(dst_hbm.at[pl.ds(off, CHUNK)], dbuf)

        def group_body(g, mcnt):
            dvec = dbuf[pl.ds(g * 16, 16)]
            msk = (dvec >= lo) & (dvec < hi)
            mi = msk.astype(jnp.int32)
            excl = plsc.cumsum(mi) - mi
            packed = ((dvec - lo) << 18) | (off + g * 16 + iota)
            plsc.store_scatter(lbuf, [excl + mcnt], packed, mask=msk)
            return mcnt + jnp.sum(mi)

        mcnt = lax.fori_loop(0, CHUNK // 16, group_body, mcnt)

        def do_flush(c):
            mc, nf = c
            foff = pl.multiple_of(nf * CHUNK, CHUNK)
            pltpu.sync_copy(lbuf.at[pl.ds(0, CHUNK)],
                            lists_hbm.at[w, pl.ds(foff, CHUNK)])

            def mv(i, _):
                v = lbuf[pl.ds(CHUNK + i * 16, 16)]
                lbuf[pl.ds(i * 16, 16)] = v
                return 0

            lax.fori_loop(0, CHUNK // 16, mv, 0)
            return (mc - CHUNK, nf + 1)

        return lax.cond(mcnt >= CHUNK, do_flush, lambda c: c, (mcnt, nflush))

    mcnt, nflush = lax.fori_loop(0, EPAD // CHUNK, chunk_body, (0, 0))
    foff = pl.multiple_of(nflush * CHUNK, CHUNK)
    pltpu.sync_copy(lbuf.at[pl.ds(0, CHUNK)], lists_hbm.at[w, pl.ds(foff, CHUNK)])
    cbuf[...] = jnp.full((16,), nflush * CHUNK + mcnt, jnp.int32)
    pltpu.sync_copy(cbuf, counts_hbm.at[w])


def _prep_call(dstp):
    return pl.kernel(
        _prep_body,
        compiler_params=pltpu.CompilerParams(needs_layout_passes=False),
        out_type=(jax.ShapeDtypeStruct((NW, CAP), jnp.int32),
                  jax.ShapeDtypeStruct((NW, 16), jnp.int32)),
        mesh=_mesh,
        scratch_types=[pltpu.VMEM((CHUNK,), jnp.int32),
                       pltpu.VMEM((2 * CHUNK + 32,), jnp.int32),
                       pltpu.VMEM((16,), jnp.int32)],
    )(dstp)


def _gather_body(table_hbm, src_hbm, g_hbm, ibuf, rows, gsem, wsem):
    w = _wid()
    base = w * (EPAD // NW)
    nb = EPAD // NW // GB
    gd = [None] * nb
    wr = [None] * nb
    for b in range(nb):
        bf = b & 1
        if b >= 2:
            wr[b - 2].wait()
        r0 = pl.multiple_of(base + b * GB, GB)
        for j in range(GB // 128):
            pltpu.sync_copy(src_hbm.at[pl.ds(r0 + j * 128, 128)],
                            ibuf.at[bf, j])
        gd[b] = [pltpu.async_copy(table_hbm.at[ibuf.at[bf, j]],
                                  rows.at[bf, pl.ds(j * 128, 128)], gsem)
                 for j in range(GB // 128)]
        if b >= 1:
            for c in gd[b - 1]:
                c.wait()
            p0 = pl.multiple_of(base + (b - 1) * GB, GB)
            wr[b - 1] = pltpu.async_copy(rows.at[(b - 1) & 1],
                                         g_hbm.at[pl.ds(p0, GB)], wsem)
    for c in gd[nb - 1]:
        c.wait()
    p0 = pl.multiple_of(base + (nb - 1) * GB, GB)
    wr[nb - 1] = pltpu.async_copy(rows.at[(nb - 1) & 1],
                                  g_hbm.at[pl.ds(p0, GB)], wsem)
    wr[nb - 2].wait()
    wr[nb - 1].wait()


def _gather_call(table, srcp):
    return pl.kernel(
        _gather_body,
        compiler_params=pltpu.CompilerParams(needs_layout_passes=False),
        out_type=jax.ShapeDtypeStruct((EPAD, 128), jnp.float32),
        mesh=_mesh,
        scratch_types=[pltpu.VMEM((2, GB // 128, 128), jnp.int32),
                       pltpu.VMEM((2, GB, 128), jnp.float32),
                       pltpu.SemaphoreType.DMA,
                       pltpu.SemaphoreType.DMA],
    )(table, srcp)


def _scat_body(m_hbm, lists_hbm, counts_hbm, mhat_hbm,
               acc, lb, dv, eidx, rows, cbuf, gsem):
    w = _wid()
    iota = _iota()
    offk = [iota + 16 * k for k in range(4)]

    def init(i, _):
        acc[pl.ds(i * 16, 16)] = jnp.full((16,), NEG, jnp.float32)
        return 0

    lax.fori_loop(0, (SH + 1) * DS // 16, init, 0)

    pltpu.sync_copy(counts_hbm.at[w], cbuf)
    nt = jnp.max(cbuf[...])
    npair = (nt + 2 * GB - 1) // (2 * GB)

    def load_stage(b, bf):
        boff = pl.multiple_of(b * GB, GB)
        pltpu.sync_copy(lists_hbm.at[w, pl.ds(boff, GB)],
                        lb.at[pl.ds(bf * GB, GB)])

        def unpack(g, _):
            pk = lb[pl.ds(bf * GB + g * 16, 16)]
            glob = b * GB + g * 16 + iota
            pk = jnp.where(glob < nt, pk, TRASH_PACKED)
            e = pk & 0x3FFFF
            dl = pk >> 18
            j2 = bf * (GB // 128) + g // 8
            c0 = (g % 8) * 16
            plsc.store_scatter(eidx, [jnp.full((16,), j2, jnp.int32),
                                      c0 + iota], e)
            dv[pl.ds(bf * GB + g * 16, 16)] = dl
            return 0

        lax.fori_loop(0, GB // 16, unpack, 0)
        return [pltpu.async_copy(m_hbm.at[eidx.at[bf * (GB // 128) + j]],
                                 rows.at[pl.ds(bf * GB + j * 128, 128)], gsem)
                for j in range(GB // 128)]

    def rmw_stage(bf):
        def rmw(g, _):
            for e in range(16):
                el = jnp.full((16,), bf * GB + g * 16 + e, jnp.int32)
                dl = plsc.load_gather(dv, [el])
                basev = dl * DS
                for k in range(4):
                    idx = basev + offk[k]
                    row = plsc.load_gather(rows, [el, offk[k]])
                    cur = plsc.load_gather(acc, [idx])
                    plsc.store_scatter(acc, [idx], jnp.maximum(cur, row))
            return 0

        lax.fori_loop(0, GB // 16, rmw, 0)

    def pair(pi, _):
        d0 = load_stage(2 * pi, 0)
        d1 = load_stage(2 * pi + 1, 1)
        for c in d0:
            c.wait()
        rmw_stage(0)
        for c in d1:
            c.wait()
        rmw_stage(1)
        return 0

    lax.fori_loop(0, npair, pair, 0)

    def fixup(i, _):
        v = acc[pl.ds(i * 16, 16)]
        acc[pl.ds(i * 16, 16)] = jnp.where(v == NEG, 0.0, v)
        return 0

    lax.fori_loop(0, SH * DS // 16, fixup, 0)
    pltpu.sync_copy(acc.at[pl.ds(0, SH * DS)],
                    mhat_hbm.at[pl.ds(w * SH * DS, SH * DS)])


def _scatter_call(m, lists, counts):
    return pl.kernel(
        _scat_body,
        compiler_params=pltpu.CompilerParams(needs_layout_passes=False),
        out_type=jax.ShapeDtypeStruct((NSEG * DS,), jnp.float32),
        mesh=_mesh,
        scratch_types=[pltpu.VMEM(((SH + 1) * DS,), jnp.float32),
                       pltpu.VMEM((2 * GB,), jnp.int32),
                       pltpu.VMEM((2 * GB,), jnp.int32),
                       pltpu.VMEM((2 * (GB // 128), 128), jnp.int32),
                       pltpu.VMEM((2 * GB, 128), jnp.float32),
                       pltpu.VMEM((16,), jnp.int32),
                       pltpu.SemaphoreType.DMA],
    )(m, lists, counts)


def kernel(state, node_feature, edge_feature, edge_index, params):
    src = edge_index[0]
    dst = edge_index[1]

    glob = jnp.broadcast_to(state[0, :5], (N, 5))
    loc1 = state[0, 5:5 + N][:, None]
    loc2 = state[0, 5 + N:5 + 2 * N][:, None]
    x13 = jnp.concatenate([glob, loc1, loc2, node_feature], axis=1)
    xpad = jnp.pad(x13, ((0, NPAD - N), (0, 3)))

    pin = params['input']
    W1in = jnp.pad(pin['W1'], ((0, 3), (0, 0)))
    h = _mlp_call(xpad, pin, W1in, DS, RB)

    pm = params['message']
    W1g = jnp.concatenate([pm['W1'][0:64], pm['W1'][65:78],
                           jnp.zeros((51, H), jnp.float32)], axis=0)
    we = pm['W1'][64:65]
    pu = params['update']

    srcp = jnp.pad(src, (0, EPAD - E))
    dstp = jnp.pad(dst, (0, EPAD - E), constant_values=1 << 20)
    efp = jnp.pad(edge_feature, (0, EPAD - E))[:, None]

    lists, counts = _prep_call(dstp)

    for _ in range(ITERS):
        table = jnp.concatenate(
            [h[:N], x13, jnp.zeros((N, 51), jnp.float32)], axis=1)
        g = _gather_call(table, srcp)
        m = _edge_mlp(g, efp, pm, W1g, we)
        mh = _scatter_call(m, lists, counts)
        m_hat = mh.reshape(NSEG, DS)[:N]
        ui = jnp.pad(jnp.concatenate([m_hat, h[:N]], axis=1),
                     ((0, NPAD - N), (0, 0)))
        h = _mlp_call(ui, pu, pu['W1'], DS, RB)

    out = _head_call(h, params['output'])
    return out[:N].reshape(N, 1, 2)
